# baseline (speedup 1.0000x reference)
import functools

import jax
import jax.numpy as jnp
from jax import lax
from jax.experimental import pallas as pl
from jax.experimental.pallas import tpu as pltpu
from jax.experimental.pallas import tpu_sc as plsc


def _make_sc_gather(V, D, B, n_workers, chunk):
    b_per_w = B // n_workers
    n_chunks = b_per_w // chunk
    mesh = plsc.VectorSubcoreMesh(core_axis_name="c", subcore_axis_name="s")

    @functools.partial(
        pl.kernel,
        mesh=mesh,
        out_type=jax.ShapeDtypeStruct((B, D), jnp.float32),
        scratch_types=[
            pltpu.VMEM((b_per_w,), jnp.int32),
            pltpu.VMEM((chunk, D), jnp.float32),
            pltpu.VMEM((chunk, D), jnp.float32),
            pltpu.SemaphoreType.DMA,
            pltpu.SemaphoreType.DMA,
        ],
    )
    def gather_kernel(tab_hbm, idx_hbm, out_hbm, idx_v, buf0, buf1, sem0, sem1):
        n_cores = 2
        wid = lax.axis_index("s") * n_cores + lax.axis_index("c")
        base = wid * b_per_w
        pltpu.sync_copy(idx_hbm.at[pl.ds(base, b_per_w)], idx_v)

        bufs = (buf0, buf1)
        sems = (sem0, sem1)
        gathers = []
        for i in range(n_chunks):
            g = pltpu.async_copy(
                tab_hbm.at[idx_v.at[pl.ds(i * chunk, chunk)]],
                bufs[i % 2],
                sems[i % 2],
            )
            gathers.append(g)
            if i >= 1:
                gathers[i - 1].wait()
                pltpu.sync_copy(
                    bufs[(i - 1) % 2],
                    out_hbm.at[pl.ds(base + (i - 1) * chunk, chunk)],
                )
        gathers[n_chunks - 1].wait()
        pltpu.sync_copy(
            bufs[(n_chunks - 1) % 2],
            out_hbm.at[pl.ds(base + (n_chunks - 1) * chunk, chunk)],
        )

    return gather_kernel


def _proj_body(nb, S, x_ref, w_ref, b_ref, o_ref):
    y = (
        jnp.dot(x_ref[...], w_ref[...], preferred_element_type=jnp.float32)
        + b_ref[...]
    )
    o_ref[...] = y.reshape(nb, S, y.shape[-1])


def _project_tokens(rows, W, b2d, Bt, S, nb):
    N, H = rows.shape
    D = W.shape[1]
    grid = (Bt // nb,)
    return pl.pallas_call(
        functools.partial(_proj_body, nb, S),
        grid=grid,
        in_specs=[
            pl.BlockSpec((nb * S, H), lambda i: (i, 0)),
            pl.BlockSpec((H, D), lambda i: (0, 0)),
            pl.BlockSpec((1, D), lambda i: (0, 0)),
        ],
        out_specs=pl.BlockSpec((nb, S, D), lambda i: (i, 0, 0)),
        out_shape=jax.ShapeDtypeStruct((Bt, S, D), jnp.float32),
    )(rows, W, b2d)


def kernel(input_ids, table, W, b):
    Bt, S = input_ids.shape
    V, H = table.shape
    D = W.shape[1]
    B = Bt * S

    idx = input_ids.reshape(B).astype(jnp.int32)
    rows = _make_sc_gather(V, H, B, n_workers=32, chunk=320)(table, idx)
    return _project_tokens(rows, W, b.reshape(1, D), Bt, S, nb=64)


# speedup vs baseline: 1.7682x; 1.7682x over previous
import functools

import jax
import jax.numpy as jnp
from jax import lax
from jax.experimental import pallas as pl
from jax.experimental.pallas import tpu as pltpu
from jax.experimental.pallas import tpu_sc as plsc


def _proj_body(t_ref, w_ref, b_ref, o_ref):
    o_ref[...] = (
        jnp.dot(t_ref[...], w_ref[...], preferred_element_type=jnp.float32)
        + b_ref[...]
    )


def _project_table(table, W, b2d, block_rows):
    V, H = table.shape
    D = W.shape[1]
    grid = (V // block_rows,)
    return pl.pallas_call(
        _proj_body,
        grid=grid,
        in_specs=[
            pl.BlockSpec((block_rows, H), lambda i: (i, 0)),
            pl.BlockSpec((H, D), lambda i: (0, 0)),
            pl.BlockSpec((1, D), lambda i: (0, 0)),
        ],
        out_specs=pl.BlockSpec((block_rows, D), lambda i: (i, 0)),
        out_shape=jax.ShapeDtypeStruct((V, D), jnp.float32),
    )(table, W, b2d)


def _make_sc_gather(V, D, B, n_workers, chunk):
    b_per_w = B // n_workers
    n_chunks = b_per_w // chunk
    mesh = plsc.VectorSubcoreMesh(core_axis_name="c", subcore_axis_name="s")

    @functools.partial(
        pl.kernel,
        mesh=mesh,
        out_type=jax.ShapeDtypeStruct((B, D), jnp.float32),
        scratch_types=[
            pltpu.VMEM((b_per_w,), jnp.int32),
            pltpu.VMEM((chunk, D), jnp.float32),
            pltpu.VMEM((chunk, D), jnp.float32),
            pltpu.SemaphoreType.DMA,
            pltpu.SemaphoreType.DMA,
        ],
    )
    def gather_kernel(tab_hbm, idx_hbm, out_hbm, idx_v, buf0, buf1, sem0, sem1):
        n_cores = 2
        wid = lax.axis_index("s") * n_cores + lax.axis_index("c")
        base = wid * b_per_w
        pltpu.sync_copy(idx_hbm.at[pl.ds(base, b_per_w)], idx_v)

        bufs = (buf0, buf1)
        sems = (sem0, sem1)
        gathers = []
        for i in range(n_chunks):
            g = pltpu.async_copy(
                tab_hbm.at[idx_v.at[pl.ds(i * chunk, chunk)]],
                bufs[i % 2],
                sems[i % 2],
            )
            gathers.append(g)
            if i >= 1:
                gathers[i - 1].wait()
                pltpu.sync_copy(
                    bufs[(i - 1) % 2],
                    out_hbm.at[pl.ds(base + (i - 1) * chunk, chunk)],
                )
        gathers[n_chunks - 1].wait()
        pltpu.sync_copy(
            bufs[(n_chunks - 1) % 2],
            out_hbm.at[pl.ds(base + (n_chunks - 1) * chunk, chunk)],
        )

    return gather_kernel


def kernel(input_ids, table, W, b):
    Bt, S = input_ids.shape
    V, H = table.shape
    D = W.shape[1]
    B = Bt * S

    proj = _project_table(table, W, b.reshape(1, D), block_rows=2000)

    idx = input_ids.T.reshape(B).astype(jnp.int32)
    flat = _make_sc_gather(V, D, B, n_workers=32, chunk=320)(proj, idx)
    return flat.reshape(S, Bt, D).transpose(1, 0, 2)


# speedup vs baseline: 2.0032x; 1.1329x over previous
import functools

import jax
import jax.numpy as jnp
from jax import lax
from jax.experimental import pallas as pl
from jax.experimental.pallas import tpu as pltpu
from jax.experimental.pallas import tpu_sc as plsc


def _proj_body(t_ref, w_ref, b_ref, o_ref):
    o_ref[...] = (
        jnp.dot(t_ref[...], w_ref[...], preferred_element_type=jnp.float32)
        + b_ref[...]
    )


def _project_table(table, W, b2d, block_rows):
    V, H = table.shape
    D = W.shape[1]
    grid = (V // block_rows,)
    return pl.pallas_call(
        _proj_body,
        grid=grid,
        in_specs=[
            pl.BlockSpec((block_rows, H), lambda i: (i, 0)),
            pl.BlockSpec((H, D), lambda i: (0, 0)),
            pl.BlockSpec((1, D), lambda i: (0, 0)),
        ],
        out_specs=pl.BlockSpec((block_rows, D), lambda i: (i, 0)),
        out_shape=jax.ShapeDtypeStruct((V, D), jnp.float32),
    )(table, W, b2d)


def _make_sc_gather(V, D, B, n_workers, chunk):
    b_per_w = B // n_workers
    n_chunks = b_per_w // chunk
    mesh = plsc.VectorSubcoreMesh(core_axis_name="c", subcore_axis_name="s")

    @functools.partial(
        pl.kernel,
        mesh=mesh,
        out_type=jax.ShapeDtypeStruct((B, D), jnp.float32),
        scratch_types=[
            pltpu.VMEM((b_per_w,), jnp.int32),
            pltpu.VMEM((chunk, D), jnp.float32),
            pltpu.VMEM((chunk, D), jnp.float32),
            pltpu.SemaphoreType.DMA,
            pltpu.SemaphoreType.DMA,
        ],
    )
    def gather_kernel(tab_hbm, idx_hbm, out_hbm, idx_v, buf0, buf1, sem0, sem1):
        n_cores = 2
        wid = lax.axis_index("s") * n_cores + lax.axis_index("c")
        base = wid * b_per_w
        pltpu.sync_copy(idx_hbm.at[pl.ds(base, b_per_w)], idx_v)

        bufs = (buf0, buf1)
        sems = (sem0, sem1)
        gathers = []
        for i in range(n_chunks):
            g = pltpu.async_copy(
                tab_hbm.at[idx_v.at[pl.ds(i * chunk, chunk)]],
                bufs[i % 2],
                sems[i % 2],
            )
            gathers.append(g)
            if i >= 1:
                gathers[i - 1].wait()
                pltpu.sync_copy(
                    bufs[(i - 1) % 2],
                    out_hbm.at[pl.ds(base + (i - 1) * chunk, chunk)],
                )
        gathers[n_chunks - 1].wait()
        pltpu.sync_copy(
            bufs[(n_chunks - 1) % 2],
            out_hbm.at[pl.ds(base + (n_chunks - 1) * chunk, chunk)],
        )

    return gather_kernel


def kernel(input_ids, table, W, b):
    Bt, S = input_ids.shape
    V, H = table.shape
    D = W.shape[1]
    B = Bt * S

    proj = _project_table(table, W, b.reshape(1, D), block_rows=5000)

    idx = input_ids.T.reshape(B).astype(jnp.int32)
    flat = _make_sc_gather(V, D, B, n_workers=32, chunk=400)(proj, idx)
    return flat.reshape(S, Bt, D).transpose(1, 0, 2)


# speedup vs baseline: 2.0759x; 1.0363x over previous
import functools

import jax
import jax.numpy as jnp
from jax import lax
from jax.experimental import pallas as pl
from jax.experimental.pallas import tpu as pltpu
from jax.experimental.pallas import tpu_sc as plsc


def _proj_body(t_ref, w_ref, b_ref, o_ref):
    o_ref[...] = (
        jnp.dot(t_ref[...], w_ref[...], preferred_element_type=jnp.float32)
        + b_ref[...]
    )


def _project_table(table, W, b2d, block_rows):
    V, H = table.shape
    D = W.shape[1]
    grid = (V // block_rows,)
    return pl.pallas_call(
        _proj_body,
        grid=grid,
        in_specs=[
            pl.BlockSpec((block_rows, H), lambda i: (i, 0)),
            pl.BlockSpec((H, D), lambda i: (0, 0)),
            pl.BlockSpec((1, D), lambda i: (0, 0)),
        ],
        out_specs=pl.BlockSpec((block_rows, D), lambda i: (i, 0)),
        out_shape=jax.ShapeDtypeStruct((V, D), jnp.float32),
    )(table, W, b2d)


def _make_sc_gather(V, D, B, n_workers, chunk):
    b_per_w = B // n_workers
    n_chunks = b_per_w // chunk
    mesh = plsc.VectorSubcoreMesh(core_axis_name="c", subcore_axis_name="s")

    @functools.partial(
        pl.kernel,
        mesh=mesh,
        out_type=jax.ShapeDtypeStruct((B, D), jnp.float32),
        scratch_types=[
            pltpu.VMEM((b_per_w,), jnp.int32),
            pltpu.VMEM((chunk, D), jnp.float32),
            pltpu.VMEM((chunk, D), jnp.float32),
            pltpu.VMEM((chunk, D), jnp.float32),
            pltpu.SemaphoreType.DMA,
            pltpu.SemaphoreType.DMA,
            pltpu.SemaphoreType.DMA,
        ],
    )
    def gather_kernel(tab_hbm, idx_hbm, out_hbm, idx_v, buf0, buf1, buf2,
                      sem0, sem1, sem2):
        n_cores = 2
        wid = lax.axis_index("s") * n_cores + lax.axis_index("c")
        base = wid * b_per_w
        pltpu.sync_copy(idx_hbm.at[pl.ds(base, b_per_w)], idx_v)

        bufs = (buf0, buf1, buf2)
        sems = (sem0, sem1, sem2)
        gathers = []
        for i in range(n_chunks):
            g = pltpu.async_copy(
                tab_hbm.at[idx_v.at[pl.ds(i * chunk, chunk)]],
                bufs[i % 3],
                sems[i % 3],
            )
            gathers.append(g)
            if i >= 2:
                gathers[i - 2].wait()
                pltpu.sync_copy(
                    bufs[(i - 2) % 3],
                    out_hbm.at[pl.ds(base + (i - 2) * chunk, chunk)],
                )
        for j in (n_chunks - 2, n_chunks - 1):
            gathers[j].wait()
            pltpu.sync_copy(
                bufs[j % 3],
                out_hbm.at[pl.ds(base + j * chunk, chunk)],
            )

    return gather_kernel


def kernel(input_ids, table, W, b):
    Bt, S = input_ids.shape
    V, H = table.shape
    D = W.shape[1]
    B = Bt * S

    proj = _project_table(table, W, b.reshape(1, D), block_rows=10000)

    idx = input_ids.T.reshape(B).astype(jnp.int32)
    flat = _make_sc_gather(V, D, B, n_workers=32, chunk=256)(proj, idx)
    return flat.reshape(S, Bt, D).transpose(1, 0, 2)
